# Initial kernel scaffold; baseline (speedup 1.0000x reference)
import jax
import jax.numpy as jnp
from jax.experimental import pallas as pl


def kernel(input):
    raise NotImplementedError("write your pallas kernel here")


# speedup vs baseline: 46.5039x; 46.5039x over previous
import functools

import jax
import jax.numpy as jnp
from jax import lax
from jax.experimental import pallas as pl
from jax.experimental.pallas import tpu as pltpu
from jax.experimental.pallas import tpu_sc as plsc

L = 16
N = 32768
NV = N // L
K = 64
ROWS = 1024
NW = 32
RPW = ROWS // NW
CAP = 128


def _lane():
    return lax.iota(jnp.int32, L)


def _vsort(x, descending):
    k, _ = plsc.sort_key_val(x, x, descending=descending)
    return k


def _bitonic_clean(vs, descending):
    q = len(vs)
    if q == 1:
        return [_vsort(vs[0], descending)]
    h = q // 2
    hi = [jnp.maximum(vs[i], vs[i + h]) for i in range(h)]
    lo = [jnp.minimum(vs[i], vs[i + h]) for i in range(h)]
    first, second = (hi, lo) if descending else (lo, hi)
    return _bitonic_clean(first, descending) + _bitonic_clean(second, descending)


def _merge_sorted(a, b, descending):
    p = len(b)
    revb = [lax.rev(b[p - 1 - i], (0,)) for i in range(p)]
    return _bitonic_clean(a + revb, descending)


def _full_sort(vs, descending):
    runs = [[_vsort(v, descending)] for v in vs]
    while len(runs) > 1:
        nxt = []
        for i in range(0, len(runs), 2):
            if i + 1 < len(runs):
                nxt.append(_merge_sorted(runs[i], runs[i + 1], descending))
            else:
                nxt.append(runs[i])
        runs = nxt
    return runs[0]


def _keybits(v, shift, lvl0):
    u = plsc.bitcast(v, jnp.int32)
    ms = jnp.bitwise_xor(u, lax.shift_right_logical(jnp.right_shift(u, 31), 1))
    b = jnp.right_shift(ms, shift)
    if lvl0:
        return b + 128
    return jnp.bitwise_and(b, 255)


UNROLL = 8


def _sc_body(x_hbm, out_hbm, data, act_a, act_b, hist, chist, acc, outrow, smem):
    lane = _lane()
    lane256 = lane * 256
    ones_i = jnp.ones((L,), jnp.int32)
    zeros_i = jnp.zeros((L,), jnp.int32)

    def hist_clear():
        def zbody(j, _):
            hist[pl.ds(j * L, L)] = zeros_i
            return 0
        lax.fori_loop(0, 256, zbody, 0)

    def hist_pass(src, n, shift, lvl0):
        nv = (n + L - 1) // L
        def body(i, _):
            v = src[pl.ds(i * L, L)]
            binq = _keybits(v, shift, lvl0)
            valid = lane < (n - i * L)
            plsc.addupdate_scatter(hist, [lane256 + binq], ones_i, mask=valid)
            return 0
        lax.fori_loop(0, nv, body, 0)

    def hist_pass_full():
        lane_off = lane256 + 128
        @plsc.parallel_loop(0, NV, unroll=UNROLL)
        def _(i):
            v = data[pl.ds(i * L, L)]
            uu = plsc.bitcast(v, jnp.int32)
            ms = jnp.bitwise_xor(
                uu, lax.shift_right_logical(jnp.right_shift(uu, 31), 1))
            sb = jnp.right_shift(ms, 24)
            plsc.addupdate_scatter(hist, [lane_off + sb], ones_i)

    def lane_counts(binv, is_top):
        def body(b, acc_v):
            return acc_v + plsc.load_gather(hist, [lane256 + b])
        if is_top:
            return lax.fori_loop(binv, 256, body, zeros_i)
        return lax.fori_loop(0, binv + 1, body, zeros_i)

    def sweep_candidates(msbound, is_top, dst, base_vec):
        msb_vec = zeros_i + msbound
        @plsc.parallel_loop(0, NV, unroll=UNROLL, carry=base_vec)
        def _(i, idxv):
            v = data[pl.ds(i * L, L)]
            uu = plsc.bitcast(v, jnp.int32)
            ms = jnp.bitwise_xor(
                uu, lax.shift_right_logical(jnp.right_shift(uu, 31), 1))
            m = (ms >= msb_vec) if is_top else (ms <= msb_vec)
            plsc.store_scatter(dst, [idxv], v, mask=m)
            return idxv + jnp.where(m, ones_i, zeros_i)

    def combine():
        def body(j, _):
            acc_v = zeros_i
            for l in range(16):
                acc_v = acc_v + hist[pl.ds(l * 256 + j * L, L)]
            chist[pl.ds(j * L, L)] = acc_v
            return 0
        lax.fori_loop(0, 16, body, 0)

    def find_threshold(target, is_top):
        def body(t, carry):
            done, binv, cstrict, run = carry
            j = (15 - t) if is_top else t
            cv = chist[pl.ds(j * L, L)]
            av = lax.rev(cv, (0,)) if is_top else cv
            cs = plsc.cumsum(av)
            tot = cs + run
            mask = tot >= target
            ts = jnp.max(plsc.all_reduce_ffs(mask))
            csm1 = jnp.sum(jnp.where(lane == ts - 1, cs, 0))
            bin_cand = (j * L + 15 - ts) if is_top else (j * L + ts)
            grp_total = jnp.sum(av)
            trig = jnp.logical_and(done == 0, ts < L)
            done2 = jnp.where(trig, jnp.int32(1), done)
            binv2 = jnp.where(trig, bin_cand, binv)
            cstrict2 = jnp.where(trig, run + csm1, cstrict)
            run2 = jnp.where(done2 > 0, run, run + grp_total)
            return done2, binv2, cstrict2, run2
        z = jnp.int32(0)
        _, binv, cstrict, _ = lax.fori_loop(0, 16, body, (z, z, z, z))
        return binv, cstrict

    def partition(src, n, binv, is_top, shift, lvl0, acc0, act_ref):
        nv = (n + L - 1) // L
        acc0_vec = jnp.full((L,), 1, jnp.int32) * acc0
        def body(i, carry):
            aoff, toff = carry
            v = src[pl.ds(i * L, L)]
            binq = _keybits(v, 24 if lvl0 else shift, lvl0)
            valid = lane < (n - i * L)
            beyond = (binq > binv) if is_top else (binq < binv)
            macc = jnp.logical_and(beyond, valid)
            mact = jnp.logical_and(binq == binv, valid)
            mi = jnp.where(macc, ones_i, zeros_i)
            cs = plsc.cumsum(mi)
            plsc.store_scatter(acc, [aoff + cs - mi], v, mask=macc)
            aoff2 = aoff + plsc.all_reduce_population_count(macc)
            mi2 = jnp.where(mact, ones_i, zeros_i)
            cs2 = plsc.cumsum(mi2)
            plsc.store_scatter(act_ref, [toff + cs2 - mi2], v, mask=mact)
            toff2 = toff + plsc.all_reduce_population_count(mact)
            return aoff2, toff2
        aoff, toff = lax.fori_loop(0, nv, body, (acc0_vec, jnp.zeros((L,), jnp.int32)))
        return jnp.max(aoff), jnp.max(toff)

    def direct_select(src, n, need, is_top, acc0):
        pad = jnp.full((L,), -jnp.inf if is_top else jnp.inf, jnp.float32)
        vs = []
        for vi in range(CAP // L):
            v = src[pl.ds(vi * L, L)]
            valid = lane < (n - vi * L)
            vs.append(jnp.where(valid, v, pad))
        svs = _full_sort(vs, descending=is_top)
        for vi in range(K // L):
            pos = lane + vi * L
            m = pos < need
            plsc.store_scatter(acc, [acc0 + pos], svs[vi], mask=m)

    def append_first(src, need, acc0):
        for vi in range(K // L):
            pos = lane + vi * L
            m = pos < need
            v = src[pl.ds(vi * L, L)]
            plsc.store_scatter(acc, [acc0 + pos], v, mask=m)

    def sort64_emit(is_top, base):
        vs = [acc[pl.ds(i * L, L)] for i in range(K // L)]
        svs = _full_sort(vs, descending=True)
        for i in range(K // L):
            outrow[pl.ds(base + i * L, L)] = svs[i]

    def run_end(is_top, binv, base, base_vec, n1):
        if is_top:
            msbound = lax.shift_left(binv - 128, 24)
        else:
            msbound = lax.shift_left(binv - 128, 24) + 0xFFFFFF
        sweep_candidates(msbound, is_top, act_a, base_vec)
        aoff, nact = partition(act_a, n1, binv, is_top, 24, True, jnp.int32(0), act_b)
        smem[0] = aoff
        smem[1] = nact
        bufs = [(act_b, act_a), (act_a, act_b), (act_b, act_a)]
        for lvl in range(1, 4):
            shift = 24 - 8 * lvl
            src, dst = bufs[lvl - 1]
            accn = smem[0]
            n = smem[1]
            need = K - accn
            @pl.when(jnp.logical_and(need > 0, n > CAP))
            def _():
                hist_clear()
                hist_pass(src, n, shift, False)
                combine()
                b2, _c2 = find_threshold(need, is_top)
                ao, na = partition(src, n, b2, is_top, shift, False, accn, dst)
                smem[0] = ao
                smem[1] = na
            @pl.when(jnp.logical_and(need > 0, n <= CAP))
            def _():
                direct_select(src, n, need, is_top, accn)
                smem[0] = jnp.int32(K)
                smem[1] = jnp.int32(0)
        need_f = K - smem[0]
        @pl.when(need_f > 0)
        def _():
            append_first(act_a, need_f, smem[0])
        sort64_emit(is_top, base)

    wid = lax.axis_index("c") * 16 + lax.axis_index("s")

    def row_body(ri, _):
        row = wid * RPW + ri
        pltpu.sync_copy(x_hbm.at[row], data)
        hist_clear()
        hist_pass_full()
        combine()
        bin_t, _cgt = find_threshold(jnp.int32(K), True)
        bin_b, _clt = find_threshold(jnp.int32(K), False)
        nv_t = lane_counts(bin_t, True)
        nv_b = lane_counts(bin_b, False)
        run_end(True, bin_t, 0, plsc.cumsum(nv_t) - nv_t, jnp.sum(nv_t))
        run_end(False, bin_b, K, plsc.cumsum(nv_b) - nv_b, jnp.sum(nv_b))
        pltpu.sync_copy(outrow, out_hbm.at[row])
        return 0

    lax.fori_loop(0, RPW, row_body, 0)


@jax.jit
def _sc_topk(x):
    mesh = plsc.VectorSubcoreMesh(
        core_axis_name="c", subcore_axis_name="s", num_cores=2, num_subcores=16
    )
    f = functools.partial(
        pl.kernel,
        out_type=jax.ShapeDtypeStruct((ROWS, 2 * K), jnp.float32),
        mesh=mesh,
        compiler_params=pltpu.CompilerParams(needs_layout_passes=False),
        scratch_types=[
            pltpu.VMEM((N,), jnp.float32),
            pltpu.VMEM((N,), jnp.float32),
            pltpu.VMEM((N,), jnp.float32),
            pltpu.VMEM((16 * 256,), jnp.int32),
            pltpu.VMEM((256,), jnp.int32),
            pltpu.VMEM((96,), jnp.float32),
            pltpu.VMEM((2 * K,), jnp.float32),
            pltpu.SMEM((8,), jnp.int32),
        ],
    )(_sc_body)
    return f(x)


def kernel(input):
    x = input.reshape(ROWS, N)
    out = _sc_topk(x)
    return out.reshape(32, 32, 2 * K)


# speedup vs baseline: 62.8400x; 1.3513x over previous
import functools

import jax
import jax.numpy as jnp
from jax import lax
from jax.experimental import pallas as pl
from jax.experimental.pallas import tpu as pltpu
from jax.experimental.pallas import tpu_sc as plsc

L = 16
N = 32768
NV = N // L
K = 64
ROWS = 1024
NW = 32
RPW = ROWS // NW
CAP = 128
CAPBIG = 1024


def _lane():
    return lax.iota(jnp.int32, L)


def _vsort(x, descending):
    k, _ = plsc.sort_key_val(x, x, descending=descending)
    return k


def _bitonic_clean(vs, descending):
    q = len(vs)
    if q == 1:
        return [_vsort(vs[0], descending)]
    h = q // 2
    hi = [jnp.maximum(vs[i], vs[i + h]) for i in range(h)]
    lo = [jnp.minimum(vs[i], vs[i + h]) for i in range(h)]
    first, second = (hi, lo) if descending else (lo, hi)
    return _bitonic_clean(first, descending) + _bitonic_clean(second, descending)


def _merge_sorted(a, b, descending):
    p = len(b)
    revb = [lax.rev(b[p - 1 - i], (0,)) for i in range(p)]
    return _bitonic_clean(a + revb, descending)


def _merge_keep_top(a, b, descending):
    p = len(a)
    revb = [lax.rev(b[p - 1 - i], (0,)) for i in range(p)]
    if descending:
        t = [jnp.maximum(a[i], revb[i]) for i in range(p)]
    else:
        t = [jnp.minimum(a[i], revb[i]) for i in range(p)]
    return _bitonic_clean(t, descending)


def _full_sort(vs, descending):
    runs = [[_vsort(v, descending)] for v in vs]
    while len(runs) > 1:
        nxt = []
        for i in range(0, len(runs), 2):
            if i + 1 < len(runs):
                nxt.append(_merge_sorted(runs[i], runs[i + 1], descending))
            else:
                nxt.append(runs[i])
        runs = nxt
    return runs[0]


def _keybits(v, shift, lvl0):
    u = plsc.bitcast(v, jnp.int32)
    ms = jnp.bitwise_xor(u, lax.shift_right_logical(jnp.right_shift(u, 31), 1))
    b = jnp.right_shift(ms, shift)
    if lvl0:
        return b + 128
    return jnp.bitwise_and(b, 255)


UNROLL = 8


NCHUNK = 8


def _sc_body(x_hbm, out_hbm, data, act_a, act_b, hist, chist, acc, outrow, smem,
             sem):
    lane = _lane()
    lane256 = lane * 256
    ones_i = jnp.ones((L,), jnp.int32)
    zeros_i = jnp.zeros((L,), jnp.int32)

    def hist_clear():
        def zbody(j, _):
            hist[pl.ds(j * L, L)] = zeros_i
            return 0
        lax.fori_loop(0, 256, zbody, 0)

    def hist_pass(src, n, shift, lvl0):
        nv = (n + L - 1) // L
        def body(i, _):
            v = src[pl.ds(i * L, L)]
            binq = _keybits(v, shift, lvl0)
            valid = lane < (n - i * L)
            plsc.addupdate_scatter(hist, [lane256 + binq], ones_i, mask=valid)
            return 0
        lax.fori_loop(0, nv, body, 0)

    def hist_chunk(c):
        lane_off = lane256 + 128
        @plsc.parallel_loop(c * (NV // NCHUNK), (c + 1) * (NV // NCHUNK),
                            unroll=UNROLL)
        def _(i):
            v = data[pl.ds(i * L, L)]
            uu = plsc.bitcast(v, jnp.int32)
            ms = jnp.bitwise_xor(
                uu, lax.shift_right_logical(jnp.right_shift(uu, 31), 1))
            sb = jnp.right_shift(ms, 24)
            plsc.addupdate_scatter(hist, [lane_off + sb], ones_i)

    def lane_counts(binv, is_top):
        def body(b, acc_v):
            return acc_v + plsc.load_gather(hist, [lane256 + b])
        if is_top:
            return lax.fori_loop(binv, 256, body, zeros_i)
        return lax.fori_loop(0, binv + 1, body, zeros_i)

    def sweep_candidates(msbound, is_top, dst, base_vec):
        msb_vec = zeros_i + msbound
        @plsc.parallel_loop(0, NV, unroll=UNROLL, carry=base_vec)
        def _(i, idxv):
            v = data[pl.ds(i * L, L)]
            uu = plsc.bitcast(v, jnp.int32)
            ms = jnp.bitwise_xor(
                uu, lax.shift_right_logical(jnp.right_shift(uu, 31), 1))
            m = (ms >= msb_vec) if is_top else (ms <= msb_vec)
            plsc.store_scatter(dst, [idxv], v, mask=m)
            return idxv + jnp.where(m, ones_i, zeros_i)

    def combine():
        def body(j, _):
            acc_v = zeros_i
            for l in range(16):
                acc_v = acc_v + hist[pl.ds(l * 256 + j * L, L)]
            chist[pl.ds(j * L, L)] = acc_v
            return 0
        lax.fori_loop(0, 16, body, 0)

    def find_threshold(target, is_top):
        def body(t, carry):
            done, binv, cstrict, run = carry
            j = (15 - t) if is_top else t
            cv = chist[pl.ds(j * L, L)]
            av = lax.rev(cv, (0,)) if is_top else cv
            cs = plsc.cumsum(av)
            tot = cs + run
            mask = tot >= target
            ts = jnp.max(plsc.all_reduce_ffs(mask))
            csm1 = jnp.sum(jnp.where(lane == ts - 1, cs, 0))
            bin_cand = (j * L + 15 - ts) if is_top else (j * L + ts)
            grp_total = jnp.sum(av)
            trig = jnp.logical_and(done == 0, ts < L)
            done2 = jnp.where(trig, jnp.int32(1), done)
            binv2 = jnp.where(trig, bin_cand, binv)
            cstrict2 = jnp.where(trig, run + csm1, cstrict)
            run2 = jnp.where(done2 > 0, run, run + grp_total)
            return done2, binv2, cstrict2, run2
        z = jnp.int32(0)
        _, binv, cstrict, _ = lax.fori_loop(0, 16, body, (z, z, z, z))
        return binv, cstrict

    def partition(src, n, binv, is_top, shift, lvl0, acc0, act_ref):
        nv = (n + L - 1) // L
        acc0_vec = jnp.full((L,), 1, jnp.int32) * acc0
        def body(i, carry):
            aoff, toff = carry
            v = src[pl.ds(i * L, L)]
            binq = _keybits(v, 24 if lvl0 else shift, lvl0)
            valid = lane < (n - i * L)
            beyond = (binq > binv) if is_top else (binq < binv)
            macc = jnp.logical_and(beyond, valid)
            mact = jnp.logical_and(binq == binv, valid)
            mi = jnp.where(macc, ones_i, zeros_i)
            cs = plsc.cumsum(mi)
            plsc.store_scatter(acc, [aoff + cs - mi], v, mask=macc)
            aoff2 = aoff + plsc.all_reduce_population_count(macc)
            mi2 = jnp.where(mact, ones_i, zeros_i)
            cs2 = plsc.cumsum(mi2)
            plsc.store_scatter(act_ref, [toff + cs2 - mi2], v, mask=mact)
            toff2 = toff + plsc.all_reduce_population_count(mact)
            return aoff2, toff2
        aoff, toff = lax.fori_loop(0, nv, body, (acc0_vec, jnp.zeros((L,), jnp.int32)))
        return jnp.max(aoff), jnp.max(toff)

    def direct_select(src, n, need, is_top, acc0):
        pad = jnp.full((L,), -jnp.inf if is_top else jnp.inf, jnp.float32)
        vs = []
        for vi in range(CAP // L):
            v = src[pl.ds(vi * L, L)]
            valid = lane < (n - vi * L)
            vs.append(jnp.where(valid, v, pad))
        svs = _full_sort(vs, descending=is_top)
        for vi in range(K // L):
            pos = lane + vi * L
            m = pos < need
            plsc.store_scatter(acc, [acc0 + pos], svs[vi], mask=m)

    def select_top_big(src, n, need, is_top, acc0):
        pad = jnp.full((L,), -jnp.inf if is_top else jnp.inf, jnp.float32)
        runs = []
        for vi in range(CAPBIG // L):
            v = src[pl.ds(vi * L, L)]
            valid = lane < (n - vi * L)
            runs.append([_vsort(jnp.where(valid, v, pad), is_top)])
        for _ in range(2):
            runs = [_merge_sorted(runs[i], runs[i + 1], is_top)
                    for i in range(0, len(runs), 2)]
        while len(runs) > 1:
            runs = [_merge_keep_top(runs[i], runs[i + 1], is_top)
                    for i in range(0, len(runs), 2)]
        svs = runs[0]
        for vi in range(K // L):
            pos = lane + vi * L
            m = pos < need
            plsc.store_scatter(acc, [acc0 + pos], svs[vi], mask=m)

    def append_first(src, need, acc0):
        for vi in range(K // L):
            pos = lane + vi * L
            m = pos < need
            v = src[pl.ds(vi * L, L)]
            plsc.store_scatter(acc, [acc0 + pos], v, mask=m)

    def sort64_emit(is_top, base):
        vs = [acc[pl.ds(i * L, L)] for i in range(K // L)]
        svs = _full_sort(vs, descending=True)
        for i in range(K // L):
            outrow[pl.ds(base + i * L, L)] = svs[i]

    def run_end(is_top, binv, base, base_vec, n1):
        if is_top:
            msbound = lax.shift_left(binv - 128, 24)
        else:
            msbound = lax.shift_left(binv - 128, 24) + 0xFFFFFF
        sweep_candidates(msbound, is_top, act_a, base_vec)
        aoff, nact = partition(act_a, n1, binv, is_top, 24, True, jnp.int32(0), act_b)
        smem[0] = aoff
        smem[1] = nact
        need0 = K - aoff
        @pl.when(jnp.logical_and(need0 > 0, nact <= CAPBIG))
        def _():
            select_top_big(act_b, nact, need0, is_top, aoff)
            smem[0] = jnp.int32(K)
            smem[1] = jnp.int32(0)
        bufs = [(act_b, act_a), (act_a, act_b), (act_b, act_a)]
        for lvl in range(1, 4):
            shift = 24 - 8 * lvl
            src, dst = bufs[lvl - 1]
            accn = smem[0]
            n = smem[1]
            need = K - accn
            @pl.when(jnp.logical_and(need > 0, n > CAP))
            def _():
                hist_clear()
                hist_pass(src, n, shift, False)
                combine()
                b2, _c2 = find_threshold(need, is_top)
                ao, na = partition(src, n, b2, is_top, shift, False, accn, dst)
                smem[0] = ao
                smem[1] = na
            if lvl > 1:
                @pl.when(jnp.logical_and(need > 0, n <= CAP))
                def _():
                    direct_select(src, n, need, is_top, accn)
                    smem[0] = jnp.int32(K)
                    smem[1] = jnp.int32(0)
        need_f = K - smem[0]
        @pl.when(need_f > 0)
        def _():
            append_first(act_a, need_f, smem[0])
        sort64_emit(is_top, base)

    wid = lax.axis_index("c") * 16 + lax.axis_index("s")

    def row_body(ri, _):
        row = wid * RPW + ri
        cw = N // NCHUNK
        cps = [pltpu.async_copy(x_hbm.at[row, pl.ds(0, cw)],
                                data.at[pl.ds(0, cw)], sem)]
        hist_clear()
        for c in range(NCHUNK):
            cps[c].wait()
            if c + 1 < NCHUNK:
                cps.append(pltpu.async_copy(
                    x_hbm.at[row, pl.ds((c + 1) * cw, cw)],
                    data.at[pl.ds((c + 1) * cw, cw)], sem))
            hist_chunk(c)
        combine()
        bin_t, _cgt = find_threshold(jnp.int32(K), True)
        bin_b, _clt = find_threshold(jnp.int32(K), False)
        nv_t = lane_counts(bin_t, True)
        nv_b = lane_counts(bin_b, False)
        run_end(True, bin_t, 0, plsc.cumsum(nv_t) - nv_t, jnp.sum(nv_t))
        run_end(False, bin_b, K, plsc.cumsum(nv_b) - nv_b, jnp.sum(nv_b))
        pltpu.sync_copy(outrow, out_hbm.at[row])
        return 0

    lax.fori_loop(0, RPW, row_body, 0)


@jax.jit
def _sc_topk(x):
    mesh = plsc.VectorSubcoreMesh(
        core_axis_name="c", subcore_axis_name="s", num_cores=2, num_subcores=16
    )
    f = functools.partial(
        pl.kernel,
        out_type=jax.ShapeDtypeStruct((ROWS, 2 * K), jnp.float32),
        mesh=mesh,
        compiler_params=pltpu.CompilerParams(needs_layout_passes=False),
        scratch_types=[
            pltpu.VMEM((N,), jnp.float32),
            pltpu.VMEM((N,), jnp.float32),
            pltpu.VMEM((N,), jnp.float32),
            pltpu.VMEM((16 * 256,), jnp.int32),
            pltpu.VMEM((256,), jnp.int32),
            pltpu.VMEM((96,), jnp.float32),
            pltpu.VMEM((2 * K,), jnp.float32),
            pltpu.SMEM((8,), jnp.int32),
            pltpu.SemaphoreType.DMA,
        ],
    )(_sc_body)
    return f(x)


def kernel(input):
    x = input.reshape(ROWS, N)
    out = _sc_topk(x)
    return out.reshape(32, 32, 2 * K)


# speedup vs baseline: 66.0510x; 1.0511x over previous
import functools

import jax
import jax.numpy as jnp
from jax import lax
from jax.experimental import pallas as pl
from jax.experimental.pallas import tpu as pltpu
from jax.experimental.pallas import tpu_sc as plsc

L = 16
N = 32768
NV = N // L
K = 64
ROWS = 1024
NW = 32
RPW = ROWS // NW
CAP = 128
CAPBIG = 1024


def _lane():
    return lax.iota(jnp.int32, L)


def _vsort(x, descending):
    k, _ = plsc.sort_key_val(x, x, descending=descending)
    return k


def _bitonic_clean(vs, descending):
    q = len(vs)
    if q == 1:
        return [_vsort(vs[0], descending)]
    h = q // 2
    hi = [jnp.maximum(vs[i], vs[i + h]) for i in range(h)]
    lo = [jnp.minimum(vs[i], vs[i + h]) for i in range(h)]
    first, second = (hi, lo) if descending else (lo, hi)
    return _bitonic_clean(first, descending) + _bitonic_clean(second, descending)


def _merge_sorted(a, b, descending):
    p = len(b)
    revb = [lax.rev(b[p - 1 - i], (0,)) for i in range(p)]
    return _bitonic_clean(a + revb, descending)


def _merge_keep_top(a, b, descending):
    p = len(a)
    revb = [lax.rev(b[p - 1 - i], (0,)) for i in range(p)]
    if descending:
        t = [jnp.maximum(a[i], revb[i]) for i in range(p)]
    else:
        t = [jnp.minimum(a[i], revb[i]) for i in range(p)]
    return _bitonic_clean(t, descending)


def _full_sort(vs, descending):
    runs = [[_vsort(v, descending)] for v in vs]
    while len(runs) > 1:
        nxt = []
        for i in range(0, len(runs), 2):
            if i + 1 < len(runs):
                nxt.append(_merge_sorted(runs[i], runs[i + 1], descending))
            else:
                nxt.append(runs[i])
        runs = nxt
    return runs[0]


def _keybits(v, shift, lvl0):
    u = plsc.bitcast(v, jnp.int32)
    ms = jnp.bitwise_xor(u, lax.shift_right_logical(jnp.right_shift(u, 31), 1))
    b = jnp.right_shift(ms, shift)
    if lvl0:
        return b + 128
    return jnp.bitwise_and(b, 255)


UNROLL = 8


NCHUNK = 8


def _sc_body(x_hbm, out_hbm, data, act_a, act_b, hist, chist, acc, outrow, smem,
             sem):
    lane = _lane()
    lane256 = lane * 256
    ones_i = jnp.ones((L,), jnp.int32)
    zeros_i = jnp.zeros((L,), jnp.int32)

    def hist_clear():
        def zbody(j, _):
            hist[pl.ds(j * L, L)] = zeros_i
            return 0
        lax.fori_loop(0, 256, zbody, 0)

    def hist_pass(src, n, shift, lvl0):
        nv = (n + L - 1) // L
        def body(i, _):
            v = src[pl.ds(i * L, L)]
            binq = _keybits(v, shift, lvl0)
            valid = lane < (n - i * L)
            plsc.addupdate_scatter(hist, [lane256 + binq], ones_i, mask=valid)
            return 0
        lax.fori_loop(0, nv, body, 0)

    def hist_chunk(c):
        lane_off = lane256 + 128
        @plsc.parallel_loop(c * (NV // NCHUNK), (c + 1) * (NV // NCHUNK),
                            unroll=UNROLL)
        def _(i):
            v = data[pl.ds(i * L, L)]
            uu = plsc.bitcast(v, jnp.int32)
            ms = jnp.bitwise_xor(
                uu, lax.shift_right_logical(jnp.right_shift(uu, 31), 1))
            sb = jnp.right_shift(ms, 24)
            plsc.addupdate_scatter(hist, [lane_off + sb], ones_i)

    def lane_counts(binv, is_top):
        def body(b, acc_v):
            return acc_v + plsc.load_gather(hist, [lane256 + b])
        if is_top:
            return lax.fori_loop(binv, 256, body, zeros_i)
        return lax.fori_loop(0, binv + 1, body, zeros_i)

    def sweep_candidates(msbound, is_top, dst, base_vec):
        msb_vec = zeros_i + msbound
        @plsc.parallel_loop(0, NV, unroll=UNROLL, carry=base_vec)
        def _(i, idxv):
            v = data[pl.ds(i * L, L)]
            uu = plsc.bitcast(v, jnp.int32)
            ms = jnp.bitwise_xor(
                uu, lax.shift_right_logical(jnp.right_shift(uu, 31), 1))
            m = (ms >= msb_vec) if is_top else (ms <= msb_vec)
            plsc.store_scatter(dst, [idxv], v, mask=m)
            return idxv + jnp.where(m, ones_i, zeros_i)

    def sweep_both(msb_t, msb_b, bt_vec, bb_vec):
        msbt_vec = zeros_i + msb_t
        msbb_vec = zeros_i + msb_b
        @plsc.parallel_loop(0, NV, unroll=UNROLL, carry=(bt_vec, bb_vec))
        def _(i, carry):
            it, ib = carry
            v = data[pl.ds(i * L, L)]
            uu = plsc.bitcast(v, jnp.int32)
            ms = jnp.bitwise_xor(
                uu, lax.shift_right_logical(jnp.right_shift(uu, 31), 1))
            mt = ms >= msbt_vec
            mb = ms <= msbb_vec
            plsc.store_scatter(act_a, [it], v, mask=mt)
            plsc.store_scatter(act_a, [ib], v, mask=mb)
            return (it + jnp.where(mt, ones_i, zeros_i),
                    ib + jnp.where(mb, ones_i, zeros_i))

    def combine():
        def body(j, _):
            acc_v = zeros_i
            for l in range(16):
                acc_v = acc_v + hist[pl.ds(l * 256 + j * L, L)]
            chist[pl.ds(j * L, L)] = acc_v
            return 0
        lax.fori_loop(0, 16, body, 0)

    def find_threshold(target, is_top):
        def body(t, carry):
            done, binv, cstrict, run = carry
            j = (15 - t) if is_top else t
            cv = chist[pl.ds(j * L, L)]
            av = lax.rev(cv, (0,)) if is_top else cv
            cs = plsc.cumsum(av)
            tot = cs + run
            mask = tot >= target
            ts = jnp.max(plsc.all_reduce_ffs(mask))
            csm1 = jnp.sum(jnp.where(lane == ts - 1, cs, 0))
            bin_cand = (j * L + 15 - ts) if is_top else (j * L + ts)
            grp_total = jnp.sum(av)
            trig = jnp.logical_and(done == 0, ts < L)
            done2 = jnp.where(trig, jnp.int32(1), done)
            binv2 = jnp.where(trig, bin_cand, binv)
            cstrict2 = jnp.where(trig, run + csm1, cstrict)
            run2 = jnp.where(done2 > 0, run, run + grp_total)
            return done2, binv2, cstrict2, run2
        z = jnp.int32(0)
        _, binv, cstrict, _ = lax.fori_loop(0, 16, body, (z, z, z, z))
        return binv, cstrict

    def partition(src, n, binv, is_top, shift, lvl0, acc0, act_ref, src_off=0):
        nv = (n + L - 1) // L
        acc0_vec = jnp.full((L,), 1, jnp.int32) * acc0
        def body(i, carry):
            aoff, toff = carry
            v = src[pl.ds(src_off + i * L, L)]
            binq = _keybits(v, 24 if lvl0 else shift, lvl0)
            valid = lane < (n - i * L)
            beyond = (binq > binv) if is_top else (binq < binv)
            macc = jnp.logical_and(beyond, valid)
            mact = jnp.logical_and(binq == binv, valid)
            mi = jnp.where(macc, ones_i, zeros_i)
            cs = plsc.cumsum(mi)
            plsc.store_scatter(acc, [aoff + cs - mi], v, mask=macc)
            aoff2 = aoff + plsc.all_reduce_population_count(macc)
            mi2 = jnp.where(mact, ones_i, zeros_i)
            cs2 = plsc.cumsum(mi2)
            plsc.store_scatter(act_ref, [toff + cs2 - mi2], v, mask=mact)
            toff2 = toff + plsc.all_reduce_population_count(mact)
            return aoff2, toff2
        aoff, toff = lax.fori_loop(0, nv, body, (acc0_vec, jnp.zeros((L,), jnp.int32)))
        return jnp.max(aoff), jnp.max(toff)

    def direct_select(src, n, need, is_top, acc0):
        pad = jnp.full((L,), -jnp.inf if is_top else jnp.inf, jnp.float32)
        vs = []
        for vi in range(CAP // L):
            v = src[pl.ds(vi * L, L)]
            valid = lane < (n - vi * L)
            vs.append(jnp.where(valid, v, pad))
        svs = _full_sort(vs, descending=is_top)
        for vi in range(K // L):
            pos = lane + vi * L
            m = pos < need
            plsc.store_scatter(acc, [acc0 + pos], svs[vi], mask=m)

    def select_top_big(src, n, need, is_top, acc0):
        pad = jnp.full((L,), -jnp.inf if is_top else jnp.inf, jnp.float32)
        runs = []
        for vi in range(CAPBIG // L):
            v = src[pl.ds(vi * L, L)]
            valid = lane < (n - vi * L)
            runs.append([_vsort(jnp.where(valid, v, pad), is_top)])
        for _ in range(2):
            runs = [_merge_sorted(runs[i], runs[i + 1], is_top)
                    for i in range(0, len(runs), 2)]
        while len(runs) > 1:
            runs = [_merge_keep_top(runs[i], runs[i + 1], is_top)
                    for i in range(0, len(runs), 2)]
        svs = runs[0]
        for vi in range(K // L):
            pos = lane + vi * L
            m = pos < need
            plsc.store_scatter(acc, [acc0 + pos], svs[vi], mask=m)

    def append_first(src, need, acc0):
        for vi in range(K // L):
            pos = lane + vi * L
            m = pos < need
            v = src[pl.ds(vi * L, L)]
            plsc.store_scatter(acc, [acc0 + pos], v, mask=m)

    def sort64_emit(is_top, base):
        vs = [acc[pl.ds(i * L, L)] for i in range(K // L)]
        svs = _full_sort(vs, descending=True)
        for i in range(K // L):
            outrow[pl.ds(base + i * L, L)] = svs[i]

    def run_end(is_top, binv, base, n1, src_off):
        aoff, nact = partition(act_a, n1, binv, is_top, 24, True, jnp.int32(0),
                               act_b, src_off=src_off)
        smem[0] = aoff
        smem[1] = nact
        need0 = K - aoff
        @pl.when(jnp.logical_and(need0 > 0, nact <= CAPBIG))
        def _():
            select_top_big(act_b, nact, need0, is_top, aoff)
            smem[0] = jnp.int32(K)
            smem[1] = jnp.int32(0)
        bufs = [(act_b, act_a), (act_a, act_b), (act_b, act_a)]
        for lvl in range(1, 4):
            shift = 24 - 8 * lvl
            src, dst = bufs[lvl - 1]
            accn = smem[0]
            n = smem[1]
            need = K - accn
            @pl.when(jnp.logical_and(need > 0, n > CAP))
            def _():
                hist_clear()
                hist_pass(src, n, shift, False)
                combine()
                b2, _c2 = find_threshold(need, is_top)
                ao, na = partition(src, n, b2, is_top, shift, False, accn, dst)
                smem[0] = ao
                smem[1] = na
            if lvl > 1:
                @pl.when(jnp.logical_and(need > 0, n <= CAP))
                def _():
                    direct_select(src, n, need, is_top, accn)
                    smem[0] = jnp.int32(K)
                    smem[1] = jnp.int32(0)
        need_f = K - smem[0]
        @pl.when(need_f > 0)
        def _():
            append_first(act_a, need_f, smem[0])
        sort64_emit(is_top, base)

    wid = lax.axis_index("c") * 16 + lax.axis_index("s")

    def row_body(ri, _):
        row = wid * RPW + ri
        cw = N // NCHUNK
        cps = [pltpu.async_copy(x_hbm.at[row, pl.ds(0, cw)],
                                data.at[pl.ds(0, cw)], sem)]
        hist_clear()
        for c in range(NCHUNK):
            cps[c].wait()
            if c + 1 < NCHUNK:
                cps.append(pltpu.async_copy(
                    x_hbm.at[row, pl.ds((c + 1) * cw, cw)],
                    data.at[pl.ds((c + 1) * cw, cw)], sem))
            hist_chunk(c)
        combine()
        bin_t, _cgt = find_threshold(jnp.int32(K), True)
        bin_b, _clt = find_threshold(jnp.int32(K), False)
        nv_t = lane_counts(bin_t, True)
        nv_b = lane_counts(bin_b, False)
        base_t = plsc.cumsum(nv_t) - nv_t
        base_b_rel = plsc.cumsum(nv_b) - nv_b
        n1_t = jnp.sum(nv_t)
        n1_b = jnp.sum(nv_b)
        msb_t = lax.shift_left(bin_t - 128, 24)
        msb_b = lax.shift_left(bin_b - 128, 24) + 0xFFFFFF
        fused = (n1_t + n1_b) <= N
        s0 = jnp.where(fused, n1_t, jnp.int32(0))
        bb_vec = base_b_rel + s0
        @pl.when(fused)
        def _():
            sweep_both(msb_t, msb_b, base_t, bb_vec)
        @pl.when(jnp.logical_not(fused))
        def _():
            sweep_candidates(msb_t, True, act_a, base_t)
        run_end(True, bin_t, 0, n1_t, jnp.int32(0))
        @pl.when(jnp.logical_not(fused))
        def _():
            sweep_candidates(msb_b, False, act_a, bb_vec)
        run_end(False, bin_b, K, n1_b, s0)
        pltpu.sync_copy(outrow, out_hbm.at[row])
        return 0

    lax.fori_loop(0, RPW, row_body, 0)


@jax.jit
def _sc_topk(x):
    mesh = plsc.VectorSubcoreMesh(
        core_axis_name="c", subcore_axis_name="s", num_cores=2, num_subcores=16
    )
    f = functools.partial(
        pl.kernel,
        out_type=jax.ShapeDtypeStruct((ROWS, 2 * K), jnp.float32),
        mesh=mesh,
        compiler_params=pltpu.CompilerParams(needs_layout_passes=False),
        scratch_types=[
            pltpu.VMEM((N,), jnp.float32),
            pltpu.VMEM((N,), jnp.float32),
            pltpu.VMEM((N,), jnp.float32),
            pltpu.VMEM((16 * 256,), jnp.int32),
            pltpu.VMEM((256,), jnp.int32),
            pltpu.VMEM((96,), jnp.float32),
            pltpu.VMEM((2 * K,), jnp.float32),
            pltpu.SMEM((8,), jnp.int32),
            pltpu.SemaphoreType.DMA,
        ],
    )(_sc_body)
    return f(x)


def kernel(input):
    x = input.reshape(ROWS, N)
    out = _sc_topk(x)
    return out.reshape(32, 32, 2 * K)


# speedup vs baseline: 74.5536x; 1.1287x over previous
import functools

import jax
import jax.numpy as jnp
from jax import lax
from jax.experimental import pallas as pl
from jax.experimental.pallas import tpu as pltpu
from jax.experimental.pallas import tpu_sc as plsc

L = 16
N = 32768
NV = N // L
K = 64
ROWS = 1024
NW = 32
RPW = ROWS // NW
CAP = 128
CAPBIG = 1024


def _lane():
    return lax.iota(jnp.int32, L)


def _vsort(x, descending):
    k, _ = plsc.sort_key_val(x, x, descending=descending)
    return k


def _bitonic_clean(vs, descending):
    q = len(vs)
    if q == 1:
        return [_vsort(vs[0], descending)]
    h = q // 2
    hi = [jnp.maximum(vs[i], vs[i + h]) for i in range(h)]
    lo = [jnp.minimum(vs[i], vs[i + h]) for i in range(h)]
    first, second = (hi, lo) if descending else (lo, hi)
    return _bitonic_clean(first, descending) + _bitonic_clean(second, descending)


def _merge_sorted(a, b, descending):
    p = len(b)
    revb = [lax.rev(b[p - 1 - i], (0,)) for i in range(p)]
    return _bitonic_clean(a + revb, descending)


def _merge_keep_top(a, b, descending):
    p = len(a)
    revb = [lax.rev(b[p - 1 - i], (0,)) for i in range(p)]
    if descending:
        t = [jnp.maximum(a[i], revb[i]) for i in range(p)]
    else:
        t = [jnp.minimum(a[i], revb[i]) for i in range(p)]
    return _bitonic_clean(t, descending)


def _full_sort(vs, descending):
    runs = [[_vsort(v, descending)] for v in vs]
    while len(runs) > 1:
        nxt = []
        for i in range(0, len(runs), 2):
            if i + 1 < len(runs):
                nxt.append(_merge_sorted(runs[i], runs[i + 1], descending))
            else:
                nxt.append(runs[i])
        runs = nxt
    return runs[0]


def _keybits(v, shift, lvl0):
    u = plsc.bitcast(v, jnp.int32)
    ms = jnp.bitwise_xor(u, lax.shift_right_logical(jnp.right_shift(u, 31), 1))
    b = jnp.right_shift(ms, shift)
    if lvl0:
        return b + 128
    return jnp.bitwise_and(b, 255)


UNROLL = 8


NCHUNK = 8


def _sc_body(x_hbm, out_hbm, data, act_a, act_b, hist, chist, acc, outrow, smem,
             sem):
    lane = _lane()
    lane256 = lane * 256
    ones_i = jnp.ones((L,), jnp.int32)
    zeros_i = jnp.zeros((L,), jnp.int32)

    def hist_clear():
        @plsc.parallel_loop(0, 256, unroll=8)
        def _(j):
            hist[pl.ds(j * L, L)] = zeros_i

    def hist_pass(src, n, shift, lvl0):
        nv = (n + L - 1) // L
        def body(i, _):
            v = src[pl.ds(i * L, L)]
            binq = _keybits(v, shift, lvl0)
            valid = lane < (n - i * L)
            plsc.addupdate_scatter(hist, [lane256 + binq], ones_i, mask=valid)
            return 0
        lax.fori_loop(0, nv, body, 0)

    def hist_chunk(c):
        lane_off = lane256 + 128
        @plsc.parallel_loop(c * (NV // NCHUNK), (c + 1) * (NV // NCHUNK),
                            unroll=UNROLL)
        def _(i):
            v = data[pl.ds(i * L, L)]
            uu = plsc.bitcast(v, jnp.int32)
            ms = jnp.bitwise_xor(
                uu, lax.shift_right_logical(jnp.right_shift(uu, 31), 1))
            sb = jnp.right_shift(ms, 24)
            plsc.addupdate_scatter(hist, [lane_off + sb], ones_i)

    def lane_counts(binv, is_top):
        def body(b, acc_v):
            return acc_v + plsc.load_gather(hist, [lane256 + b])
        if is_top:
            return lax.fori_loop(binv, 256, body, zeros_i)
        return lax.fori_loop(0, binv + 1, body, zeros_i)

    def sweep_candidates(msbound, is_top, dst, base_vec):
        msb_vec = zeros_i + msbound
        @plsc.parallel_loop(0, NV, unroll=UNROLL, carry=base_vec)
        def _(i, idxv):
            v = data[pl.ds(i * L, L)]
            uu = plsc.bitcast(v, jnp.int32)
            ms = jnp.bitwise_xor(
                uu, lax.shift_right_logical(jnp.right_shift(uu, 31), 1))
            m = (ms >= msb_vec) if is_top else (ms <= msb_vec)
            plsc.store_scatter(dst, [idxv], v, mask=m)
            return idxv + jnp.where(m, ones_i, zeros_i)

    def sweep_both(msb_t, msb_b, bt_vec, bb_vec):
        msbt_vec = zeros_i + msb_t
        msbb_vec = zeros_i + msb_b
        @plsc.parallel_loop(0, NV, unroll=UNROLL, carry=(bt_vec, bb_vec))
        def _(i, carry):
            it, ib = carry
            v = data[pl.ds(i * L, L)]
            uu = plsc.bitcast(v, jnp.int32)
            ms = jnp.bitwise_xor(
                uu, lax.shift_right_logical(jnp.right_shift(uu, 31), 1))
            mt = ms >= msbt_vec
            mb = ms <= msbb_vec
            plsc.store_scatter(act_a, [it], v, mask=mt)
            plsc.store_scatter(act_a, [ib], v, mask=mb)
            return (it + jnp.where(mt, ones_i, zeros_i),
                    ib + jnp.where(mb, ones_i, zeros_i))

    def sweep_both_float(f_lo, f_hi, bt_vec, bb_vec):
        @plsc.parallel_loop(0, NV, unroll=UNROLL, carry=(bt_vec, bb_vec))
        def _(i, carry):
            it, ib = carry
            v = data[pl.ds(i * L, L)]
            mt = v >= f_lo
            mb = v <= f_hi
            plsc.store_scatter(act_a, [it], v, mask=mt)
            plsc.store_scatter(act_a, [ib], v, mask=mb)
            return (it + jnp.where(mt, ones_i, zeros_i),
                    ib + jnp.where(mb, ones_i, zeros_i))

    def combine():
        def body(j, _):
            acc_v = zeros_i
            for l in range(16):
                acc_v = acc_v + hist[pl.ds(l * 256 + j * L, L)]
            chist[pl.ds(j * L, L)] = acc_v
            return 0
        lax.fori_loop(0, 16, body, 0)

    def find_threshold(target, is_top):
        def body(t, carry):
            done, binv, cstrict, run = carry
            j = (15 - t) if is_top else t
            cv = chist[pl.ds(j * L, L)]
            av = lax.rev(cv, (0,)) if is_top else cv
            cs = plsc.cumsum(av)
            tot = cs + run
            mask = tot >= target
            ts = jnp.max(plsc.all_reduce_ffs(mask))
            csm1 = jnp.sum(jnp.where(lane == ts - 1, cs, 0))
            bin_cand = (j * L + 15 - ts) if is_top else (j * L + ts)
            grp_total = jnp.sum(av)
            trig = jnp.logical_and(done == 0, ts < L)
            done2 = jnp.where(trig, jnp.int32(1), done)
            binv2 = jnp.where(trig, bin_cand, binv)
            cstrict2 = jnp.where(trig, run + csm1, cstrict)
            run2 = jnp.where(done2 > 0, run, run + grp_total)
            return done2, binv2, cstrict2, run2
        z = jnp.int32(0)
        _, binv, cstrict, _ = lax.fori_loop(0, 16, body, (z, z, z, z))
        return binv, cstrict

    def partition(src, n, binv, is_top, shift, lvl0, acc0, act_ref, src_off=0):
        nv = (n + L - 1) // L
        acc0_vec = jnp.full((L,), 1, jnp.int32) * acc0
        @plsc.parallel_loop(0, nv, unroll=4,
                            carry=(acc0_vec, jnp.zeros((L,), jnp.int32)))
        def body(i, carry):
            aoff, toff = carry
            v = src[pl.ds(src_off + i * L, L)]
            binq = _keybits(v, 24 if lvl0 else shift, lvl0)
            valid = lane < (n - i * L)
            beyond = (binq > binv) if is_top else (binq < binv)
            macc = jnp.logical_and(beyond, valid)
            mact = jnp.logical_and(binq == binv, valid)
            mi = jnp.where(macc, ones_i, zeros_i)
            cs = plsc.cumsum(mi)
            plsc.store_scatter(acc, [aoff + cs - mi], v, mask=macc)
            aoff2 = aoff + plsc.all_reduce_population_count(macc)
            mi2 = jnp.where(mact, ones_i, zeros_i)
            cs2 = plsc.cumsum(mi2)
            plsc.store_scatter(act_ref, [toff + cs2 - mi2], v, mask=mact)
            toff2 = toff + plsc.all_reduce_population_count(mact)
            return aoff2, toff2
        aoff, toff = body
        return jnp.max(aoff), jnp.max(toff)

    def direct_select(src, n, need, is_top, acc0):
        pad = jnp.full((L,), -jnp.inf if is_top else jnp.inf, jnp.float32)
        vs = []
        for vi in range(CAP // L):
            v = src[pl.ds(vi * L, L)]
            valid = lane < (n - vi * L)
            vs.append(jnp.where(valid, v, pad))
        svs = _full_sort(vs, descending=is_top)
        for vi in range(K // L):
            pos = lane + vi * L
            m = pos < need
            plsc.store_scatter(acc, [acc0 + pos], svs[vi], mask=m)

    def select_top_big(src, n, need, is_top, acc0):
        pad = jnp.full((L,), -jnp.inf if is_top else jnp.inf, jnp.float32)
        runs = []
        for vi in range(CAPBIG // L):
            v = src[pl.ds(vi * L, L)]
            valid = lane < (n - vi * L)
            runs.append([_vsort(jnp.where(valid, v, pad), is_top)])
        for _ in range(2):
            runs = [_merge_sorted(runs[i], runs[i + 1], is_top)
                    for i in range(0, len(runs), 2)]
        while len(runs) > 1:
            runs = [_merge_keep_top(runs[i], runs[i + 1], is_top)
                    for i in range(0, len(runs), 2)]
        svs = runs[0]
        for vi in range(K // L):
            pos = lane + vi * L
            m = pos < need
            plsc.store_scatter(acc, [acc0 + pos], svs[vi], mask=m)

    def append_first(src, need, acc0):
        for vi in range(K // L):
            pos = lane + vi * L
            m = pos < need
            v = src[pl.ds(vi * L, L)]
            plsc.store_scatter(acc, [acc0 + pos], v, mask=m)

    def sort64_emit(is_top, base):
        vs = [acc[pl.ds(i * L, L)] for i in range(K // L)]
        svs = _full_sort(vs, descending=True)
        for i in range(K // L):
            outrow[pl.ds(base + i * L, L)] = svs[i]

    def run_end(is_top, binv, base, n1, src_off):
        aoff, nact = partition(act_a, n1, binv, is_top, 24, True, jnp.int32(0),
                               act_b, src_off=src_off)
        smem[0] = aoff
        smem[1] = nact
        need0 = K - aoff
        @pl.when(jnp.logical_and(need0 > 0, nact <= CAPBIG))
        def _():
            select_top_big(act_b, nact, need0, is_top, aoff)
            smem[0] = jnp.int32(K)
            smem[1] = jnp.int32(0)
        bufs = [(act_b, act_a), (act_a, act_b), (act_b, act_a)]
        for lvl in range(1, 4):
            shift = 24 - 8 * lvl
            src, dst = bufs[lvl - 1]
            accn = smem[0]
            n = smem[1]
            need = K - accn
            @pl.when(jnp.logical_and(need > 0, n > CAP))
            def _():
                hist_clear()
                hist_pass(src, n, shift, False)
                combine()
                b2, _c2 = find_threshold(need, is_top)
                ao, na = partition(src, n, b2, is_top, shift, False, accn, dst)
                smem[0] = ao
                smem[1] = na
            if lvl > 1:
                @pl.when(jnp.logical_and(need > 0, n <= CAP))
                def _():
                    direct_select(src, n, need, is_top, accn)
                    smem[0] = jnp.int32(K)
                    smem[1] = jnp.int32(0)
        need_f = K - smem[0]
        @pl.when(need_f > 0)
        def _():
            append_first(act_a, need_f, smem[0])
        sort64_emit(is_top, base)

    wid = lax.axis_index("c") * 16 + lax.axis_index("s")

    def row_body(ri, _):
        row = wid * RPW + ri
        cw = N // NCHUNK
        cps = [pltpu.async_copy(x_hbm.at[row, pl.ds(0, cw)],
                                data.at[pl.ds(0, cw)], sem)]
        hist_clear()
        for c in range(NCHUNK):
            cps[c].wait()
            if c + 1 < NCHUNK:
                cps.append(pltpu.async_copy(
                    x_hbm.at[row, pl.ds((c + 1) * cw, cw)],
                    data.at[pl.ds((c + 1) * cw, cw)], sem))
            hist_chunk(c)
        combine()
        bin_t, _cgt = find_threshold(jnp.int32(K), True)
        bin_b, _clt = find_threshold(jnp.int32(K), False)
        nv_t = lane_counts(bin_t, True)
        nv_b = lane_counts(bin_b, False)
        base_t = plsc.cumsum(nv_t) - nv_t
        base_b_rel = plsc.cumsum(nv_b) - nv_b
        n1_t = jnp.sum(nv_t)
        n1_b = jnp.sum(nv_b)
        msb_t = lax.shift_left(bin_t - 128, 24)
        msb_b = lax.shift_left(bin_b - 128, 24) + 0xFFFFFF
        fused = (n1_t + n1_b) <= N
        s0 = jnp.where(fused, n1_t, jnp.int32(0))
        bb_vec = base_b_rel + s0
        float_ok = jnp.logical_and(msb_t != 0, msb_b != -1)
        def inv_ms(msv):
            return plsc.bitcast(
                jnp.bitwise_xor(
                    msv, lax.shift_right_logical(jnp.right_shift(msv, 31), 1)),
                jnp.float32)
        f_lo = inv_ms(zeros_i + msb_t)
        f_hi = inv_ms(zeros_i + msb_b)
        @pl.when(jnp.logical_and(fused, float_ok))
        def _():
            sweep_both_float(f_lo, f_hi, base_t, bb_vec)
        @pl.when(jnp.logical_and(fused, jnp.logical_not(float_ok)))
        def _():
            sweep_both(msb_t, msb_b, base_t, bb_vec)
        @pl.when(jnp.logical_not(fused))
        def _():
            sweep_candidates(msb_t, True, act_a, base_t)
        run_end(True, bin_t, 0, n1_t, jnp.int32(0))
        @pl.when(jnp.logical_not(fused))
        def _():
            sweep_candidates(msb_b, False, act_a, bb_vec)
        run_end(False, bin_b, K, n1_b, s0)
        pltpu.sync_copy(outrow, out_hbm.at[row])
        return 0

    lax.fori_loop(0, RPW, row_body, 0)


@jax.jit
def _sc_topk(x):
    mesh = plsc.VectorSubcoreMesh(
        core_axis_name="c", subcore_axis_name="s", num_cores=2, num_subcores=16
    )
    f = functools.partial(
        pl.kernel,
        out_type=jax.ShapeDtypeStruct((ROWS, 2 * K), jnp.float32),
        mesh=mesh,
        compiler_params=pltpu.CompilerParams(needs_layout_passes=False),
        scratch_types=[
            pltpu.VMEM((N,), jnp.float32),
            pltpu.VMEM((N,), jnp.float32),
            pltpu.VMEM((N,), jnp.float32),
            pltpu.VMEM((16 * 256,), jnp.int32),
            pltpu.VMEM((256,), jnp.int32),
            pltpu.VMEM((96,), jnp.float32),
            pltpu.VMEM((2 * K,), jnp.float32),
            pltpu.SMEM((8,), jnp.int32),
            pltpu.SemaphoreType.DMA,
        ],
    )(_sc_body)
    return f(x)


def kernel(input):
    x = input.reshape(ROWS, N)
    out = _sc_topk(x)
    return out.reshape(32, 32, 2 * K)
